# baseline (speedup 1.0000x reference)
import functools

import jax
import jax.numpy as jnp
from jax import lax
from jax.experimental import pallas as pl
from jax.experimental.pallas import tpu as pltpu
from jax.experimental.pallas import tpu_sc as plsc


_NS = 16
_ZB = 40
_SB = 40


@functools.lru_cache(maxsize=None)
def _make_unpool(N: int, M: int, D: int):
  assert N % _ZB == 0 and M % _SB == 0 and D % 16 == 0
  n_zero_blocks = N // _ZB
  n_sc_blocks = M // _SB
  zero_iters = -(-n_zero_blocks // _NS)
  sc_iters = -(-n_sc_blocks // _NS)

  mesh = plsc.VectorSubcoreMesh(
      core_axis_name="c", subcore_axis_name="s", num_cores=1)

  @functools.partial(
      pl.kernel,
      mesh=mesh,
      out_type=jax.ShapeDtypeStruct((N, D), jnp.float32),
      scratch_types=[
          pltpu.VMEM((_ZB, D), jnp.float32),
          pltpu.VMEM((_SB,), jnp.int32),
          pltpu.VMEM((_SB, D), jnp.float32),
          pltpu.SemaphoreType.DMA,
          pltpu.SemaphoreType.DMA,
      ],
  )
  def unpool(x_hbm, idx_hbm, out_hbm, zb, idx_v, x_v, zsem, ssem):
    tid = lax.axis_index("s")

    z16 = jnp.zeros((16,), jnp.float32)

    @pl.loop(0, _ZB)
    def _(i):
      @pl.loop(0, D // 16)
      def _(j):
        zb[i, pl.ds(j * 16, 16)] = z16

    @pl.loop(0, zero_iters)
    def _(k):
      b = tid + k * _NS

      @pl.when(b < n_zero_blocks)
      def _():
        pltpu.async_copy(zb, out_hbm.at[pl.ds(b * _ZB, _ZB)], zsem)

    @pl.loop(0, zero_iters)
    def _(k):
      b = tid + k * _NS

      @pl.when(b < n_zero_blocks)
      def _():
        pltpu.make_async_copy(
            zb, out_hbm.at[pl.ds(b * _ZB, _ZB)], zsem).wait()

    plsc.subcore_barrier()

    @pl.loop(0, sc_iters)
    def _(k):
      b = tid + k * _NS

      @pl.when(b < n_sc_blocks)
      def _():
        pltpu.async_copy(idx_hbm.at[pl.ds(b * _SB, _SB)], idx_v, ssem)
        pltpu.async_copy(x_hbm.at[pl.ds(b * _SB, _SB)], x_v, ssem)
        pltpu.make_async_copy(
            idx_hbm.at[pl.ds(b * _SB, _SB)], idx_v, ssem).wait()
        pltpu.make_async_copy(
            x_hbm.at[pl.ds(b * _SB, _SB)], x_v, ssem).wait()
        pltpu.async_copy(x_v, out_hbm.at[idx_v], ssem).wait()

  return unpool


def kernel(A, X, idx):
  M, D = X.shape
  N = A.shape[0]
  new_X = _make_unpool(N, M, D)(X, idx.astype(jnp.int32))
  return (A, new_X)
